# baseline (speedup 1.0000x reference)
import jax
import jax.numpy as jnp
from jax.experimental import pallas as pl
from jax.experimental.pallas import tpu as pltpu

BATCH = 1024
DIM_IN = 256
DIM_OUT = 256
NUM_X = 64
KB = 16
NSTEPS = NUM_X // KB


def _interp_matmul_kernel(params_ref, x_ref, yn_ref, out_ref):
    s = pl.program_id(0)
    xmin = params_ref[0, 0]
    inv_h = params_ref[0, 1]
    u = (x_ref[...] - xmin) * inv_h
    uc = jnp.clip(u, 0.0, float(NUM_X - 1))
    base = (s * KB).astype(jnp.float32)

    mslices = []
    for j in range(0, KB, 2):
        d0 = (uc - (base + float(j))).astype(jnp.bfloat16)
        d1 = d0 - jnp.bfloat16(1.0)
        mslices.append(jnp.clip(jnp.abs(d0), jnp.bfloat16(0.0), jnp.bfloat16(1.0)))
        mslices.append(jnp.clip(jnp.abs(d1), jnp.bfloat16(0.0), jnp.bfloat16(1.0)))
    mm = jnp.concatenate(mslices, axis=1)

    acc = jax.lax.dot_general(
        mm,
        yn_ref[...].reshape(KB * DIM_IN, DIM_OUT),
        (((1,), (0,)), ((), ())),
        preferred_element_type=jnp.float32,
    )
    arow = -jnp.sum(yn_ref[...].astype(jnp.float32), axis=(0, 1))
    step_out = acc + arow[None, :]

    @pl.when(s == 0)
    def _first():
        e0 = jnp.minimum(u, 0.0).astype(jnp.bfloat16)
        d0 = yn_ref[0] - yn_ref[1]
        corr = jax.lax.dot_general(
            e0, d0, (((1,), (0,)), ((), ())),
            preferred_element_type=jnp.float32,
        )
        out_ref[...] = step_out + corr

    @pl.when(jnp.logical_and(s > 0, s < NSTEPS - 1))
    def _mid():
        out_ref[...] += step_out

    @pl.when(s == NSTEPS - 1)
    def _last():
        e1 = jnp.maximum(u - float(NUM_X - 1), 0.0).astype(jnp.bfloat16)
        d1 = yn_ref[KB - 2] - yn_ref[KB - 1]
        corr = jax.lax.dot_general(
            e1, d1, (((1,), (0,)), ((), ())),
            preferred_element_type=jnp.float32,
        )
        out_ref[...] += step_out + corr


@jax.jit
def kernel(x, X, Y):
    xmin = X[0]
    inv_h = (NUM_X - 1) / (X[NUM_X - 1] - X[0])
    params = jnp.stack([xmin, inv_h]).reshape(1, 2)
    yneg = (-jnp.transpose(Y, (2, 0, 1))).astype(jnp.bfloat16)

    out = pl.pallas_call(
        _interp_matmul_kernel,
        grid=(NSTEPS,),
        in_specs=[
            pl.BlockSpec(memory_space=pltpu.SMEM),
            pl.BlockSpec((BATCH, DIM_IN), lambda s: (0, 0)),
            pl.BlockSpec((KB, DIM_IN, DIM_OUT), lambda s: (s, 0, 0)),
        ],
        out_specs=pl.BlockSpec((BATCH, DIM_OUT), lambda s: (0, 0)),
        out_shape=jax.ShapeDtypeStruct((BATCH, DIM_OUT), jnp.float32),
    )(params, x, yneg)
    return out
